# baseline (speedup 1.0000x reference)
import functools

import numpy as np
import jax
import jax.numpy as jnp
from jax.experimental import pallas as pl

_NUM_BINS = 15
_LOG2E = float(np.log2(np.e))
_BOUNDS = np.linspace(0.0, 1.0, _NUM_BINS + 1, dtype=np.float32)
_BND = np.zeros((16, 128), dtype=np.float32)
_BND[:, 0] = 2.0
_BND[:, 1] = 3.0
_BND[:_NUM_BINS, 0] = _BOUNDS[:-1]
_BND[:_NUM_BINS, 1] = _BOUNDS[1:]


def _ece_body(x_ref, lbl_ref, bnd_ref, out_ref, *, nsteps, n_total, ncls, blk):
    i = pl.program_id(0)

    @pl.when(i == 0)
    def _init():
        out_ref[...] = jnp.zeros_like(out_ref)

    xt = x_ref[...]
    lbl = lbl_ref[0, :, :]

    e = jnp.exp2(xt * _LOG2E)
    s = jnp.sum(e, axis=0, keepdims=True)
    emax = jnp.max(e, axis=0, keepdims=True)
    row_iota = jax.lax.broadcasted_iota(jnp.int32, xt.shape, 0)
    e_lbl = jnp.max(jnp.where(row_iota == lbl, e, 0.0), axis=0, keepdims=True)

    conf = emax * jax.lax.reciprocal(s)
    accb = e_lbl == emax
    accf = jnp.where(accb, 1.0, 0.0)

    lo = bnd_ref[:, 0:1]
    up = bnd_ref[:, 1:2]
    ohb = (conf > lo) & (conf <= up)
    cnt = jnp.sum(jnp.where(ohb, 1.0, 0.0), axis=1, keepdims=True)
    sc = jnp.sum(jnp.where(ohb, conf, 0.0), axis=1, keepdims=True)
    sa = jnp.sum(jnp.where(ohb, accf, 0.0), axis=1, keepdims=True)

    out_ref[0:16, 0:1] += cnt
    out_ref[0:16, 1:2] += sc
    out_ref[0:16, 2:3] += sa

    @pl.when(i == nsteps - 1)
    def _finish():
        cntv = out_ref[0:16, 0:1]
        scv = out_ref[0:16, 1:2]
        sav = out_ref[0:16, 2:3]
        denom = jnp.maximum(cntv, 1.0)
        prop = cntv / n_total
        per = jnp.where(cntv > 0,
                        jnp.abs(scv / denom - sav / denom) * prop, 0.0)
        out_ref[0:16, 3:4] = jnp.broadcast_to(jnp.sum(per), (16, 1))


def kernel(logits, labels):
    n, c = logits.shape
    blk = 32768
    nsteps = (n + blk - 1) // blk
    npad = nsteps * blk
    cp = c
    xt_full = jnp.pad(
        jnp.transpose(logits), ((0, cp - c), (0, npad - n)),
        constant_values=-1e30,
    )
    lbl3 = jnp.pad(labels.astype(jnp.int32),
                   (0, nsteps * blk - n)).reshape(nsteps, 1, blk)
    out = pl.pallas_call(
        functools.partial(_ece_body, nsteps=nsteps, n_total=float(n), ncls=c,
                          blk=blk),
        grid=(nsteps,),
        in_specs=[
            pl.BlockSpec((cp, blk), lambda i: (0, i)),
            pl.BlockSpec((1, 1, blk), lambda i: (i, 0, 0)),
            pl.BlockSpec((16, 128), lambda i: (0, 0)),
        ],
        out_specs=pl.BlockSpec((16, 128), lambda i: (0, 0)),
        out_shape=jax.ShapeDtypeStruct((16, 128), jnp.float32),
    )(xt_full, lbl3, jnp.asarray(_BND))
    return out[0, 3].reshape(1)


# speedup vs baseline: 1.0129x; 1.0129x over previous
import functools

import numpy as np
import jax
import jax.numpy as jnp
from jax.experimental import pallas as pl

_NUM_BINS = 15
_LOG2E = float(np.log2(np.e))
_BOUNDS = np.linspace(0.0, 1.0, _NUM_BINS + 1, dtype=np.float32)
_BND = np.zeros((16, 128), dtype=np.float32)
_BND[:, 0] = 2.0
_BND[:, 1] = 3.0
_BND[:_NUM_BINS, 0] = _BOUNDS[:-1]
_BND[:_NUM_BINS, 1] = _BOUNDS[1:]


def _ece_body(x_ref, lbl_ref, bnd_ref, out_ref, *, nsteps, n_total, ncls, blk):
    i = pl.program_id(0)

    @pl.when(i == 0)
    def _init():
        out_ref[...] = jnp.zeros_like(out_ref)

    xt = x_ref[...]
    lbl = lbl_ref[0, :, :]

    e = jnp.exp2(xt * _LOG2E)
    s = jnp.sum(e, axis=0, keepdims=True)
    emax = jnp.max(e, axis=0, keepdims=True)
    row_iota = jax.lax.broadcasted_iota(jnp.int32, xt.shape, 0)
    e_lbl = jnp.max(jnp.where(row_iota == lbl, e, 0.0), axis=0, keepdims=True)

    conf = emax * jax.lax.reciprocal(s)
    accb = e_lbl == emax
    accf = jnp.where(accb, 1.0, 0.0)

    lo = bnd_ref[:, 0:1]
    up = bnd_ref[:, 1:2]
    ohb = (conf > lo) & (conf <= up)
    cnt = jnp.sum(jnp.where(ohb, 1.0, 0.0), axis=1, keepdims=True)
    sc = jnp.sum(jnp.where(ohb, conf, 0.0), axis=1, keepdims=True)
    sa = jnp.sum(jnp.where(ohb, accf, 0.0), axis=1, keepdims=True)

    out_ref[0:16, 0:1] += cnt
    out_ref[0:16, 1:2] += sc
    out_ref[0:16, 2:3] += sa

    @pl.when(i == nsteps - 1)
    def _finish():
        cntv = out_ref[0:16, 0:1]
        scv = out_ref[0:16, 1:2]
        sav = out_ref[0:16, 2:3]
        denom = jnp.maximum(cntv, 1.0)
        prop = cntv / n_total
        per = jnp.where(cntv > 0,
                        jnp.abs(scv / denom - sav / denom) * prop, 0.0)
        out_ref[0:16, 3:4] = jnp.broadcast_to(jnp.sum(per), (16, 1))


def kernel(logits, labels):
    n, c = logits.shape
    blk = 32768
    nsteps = (n + blk - 1) // blk
    npad = nsteps * blk
    cp = (c + 7) // 8 * 8
    xt_full = jnp.pad(
        jnp.transpose(logits), ((0, cp - c), (0, npad - n)),
        constant_values=-1e30,
    )
    lbl3 = jnp.pad(labels.astype(jnp.int32),
                   (0, nsteps * blk - n)).reshape(nsteps, 1, blk)
    out = pl.pallas_call(
        functools.partial(_ece_body, nsteps=nsteps, n_total=float(n), ncls=c,
                          blk=blk),
        grid=(nsteps,),
        in_specs=[
            pl.BlockSpec((cp, blk), lambda i: (0, i)),
            pl.BlockSpec((1, 1, blk), lambda i: (i, 0, 0)),
            pl.BlockSpec((16, 128), lambda i: (0, 0)),
        ],
        out_specs=pl.BlockSpec((16, 128), lambda i: (0, 0)),
        out_shape=jax.ShapeDtypeStruct((16, 128), jnp.float32),
    )(xt_full, lbl3, jnp.asarray(_BND))
    return out[0, 3].reshape(1)
